# Initial kernel scaffold; baseline (speedup 1.0000x reference)
import jax
import jax.numpy as jnp
from jax.experimental import pallas as pl


def kernel(user_embedding, item_embedding, graph_values, edge_index, users, positive_items, negative_items):
    raise NotImplementedError("write your pallas kernel here")


# speedup vs baseline: 10.4375x; 10.4375x over previous
import functools

import jax
import jax.numpy as jnp
from jax import lax
from jax.experimental import pallas as pl
from jax.experimental.pallas import tpu as pltpu
from jax.experimental.pallas import tpu_sc as plsc

_NUM_USERS = 50000
_N = 100000
_EMB = 16
_E = 3200000
_BATCH = 4096
_N_LAYERS = 3

_NC = 2
_NS = 16
_BLK = 128
_NBLK = _E // _BLK
_BLK_PER_CORE = _NBLK // _NC
_TILE_ITERS = -(-_BLK_PER_CORE // _NS)
_ROWS_PER_CORE = _N // _NC
_ROWS_PER_TILE = _ROWS_PER_CORE // _NS
_ZROWS = 125


def _layer_body(x_hbm, rows_hbm, cols_hbm, vals_hbm, out_hbm,
                acc, cidx, ridx, vals_v, gbuf, zbuf):
  c = lax.axis_index("c")
  s = lax.axis_index("s")

  for j in range(_ZROWS):
    zbuf[j, :] = jnp.zeros((_EMB,), jnp.float32)
  row_base = c * _ROWS_PER_CORE + s * _ROWS_PER_TILE
  for i in range(_ROWS_PER_TILE // _ZROWS):
    pltpu.sync_copy(zbuf, acc.at[pl.ds(row_base + i * _ZROWS, _ZROWS)])
  plsc.subcore_barrier()

  blk_core0 = c * _BLK_PER_CORE

  def step(t, carry):
    blk_local = t * _NS + s

    @pl.when(blk_local < _BLK_PER_CORE)
    def _():
      blk = blk_core0 + blk_local
      pltpu.sync_copy(cols_hbm.at[blk], cidx)
      pltpu.sync_copy(rows_hbm.at[blk], ridx)
      pltpu.sync_copy(vals_hbm.at[blk], vals_v)
      pltpu.sync_copy(x_hbm.at[cidx], gbuf)

      def scale(g, carry2):
        vals16 = vals_v[pl.ds(g * 16, 16)]
        for l in range(16):
          j = g * 16 + l
          gbuf[j, :] = gbuf[j, :] * vals16[l]
        return carry2

      lax.fori_loop(0, _BLK // 16, scale, 0)
      pltpu.sync_copy(gbuf, acc.at[ridx], add=True)

    return carry

  lax.fori_loop(0, _TILE_ITERS, step, 0)
  plsc.subcore_barrier()
  pltpu.sync_copy(acc.at[pl.ds(row_base, _ROWS_PER_TILE)],
                  out_hbm.at[pl.ds(row_base, _ROWS_PER_TILE)])


_layer_kernel = functools.partial(
    pl.kernel,
    out_type=jax.ShapeDtypeStruct((_N, _EMB), jnp.float32),
    compiler_params=pltpu.CompilerParams(use_tc_tiling_on_sc=False),
    mesh=plsc.VectorSubcoreMesh(core_axis_name="c", subcore_axis_name="s"),
    scratch_types=[
        pltpu.VMEM_SHARED((_N, _EMB), jnp.float32),
        pltpu.VMEM((_BLK,), jnp.int32),
        pltpu.VMEM((_BLK,), jnp.int32),
        pltpu.VMEM((_BLK,), jnp.float32),
        pltpu.VMEM((_BLK, _EMB), jnp.float32),
        pltpu.VMEM((_ZROWS, _EMB), jnp.float32),
    ],
)(_layer_body)


_BPW = _BATCH // (_NC * _NS)


def _final_body(x0, x1, x2, x3, users_hbm, pos_hbm, neg_hbm, out_hbm,
                uidx, pidx, nidx, ua, pa, na, gtmp, prod):
  c = lax.axis_index("c")
  s = lax.axis_index("s")
  w = c * _NS + s

  pltpu.sync_copy(users_hbm.at[w], uidx)
  pltpu.sync_copy(pos_hbm.at[w], pidx)
  pltpu.sync_copy(neg_hbm.at[w], nidx)
  for i in range(_BPW // 16):
    sl = pl.ds(i * 16, 16)
    pidx[sl] = pidx[sl] + _NUM_USERS
    nidx[sl] = nidx[sl] + _NUM_USERS

  for idx, accv in ((uidx, ua), (pidx, pa), (nidx, na)):
    pltpu.sync_copy(x0.at[idx], accv)
    for x in (x1, x2, x3):
      pltpu.sync_copy(x.at[idx], gtmp)

      def addloop(j, carry, accv=accv):
        accv[j, :] = accv[j, :] + gtmp[j, :]
        return carry

      lax.fori_loop(0, _BPW, addloop, 0)

  def prodloop(j, carry):
    prod[j, :] = ua[j, :] * (na[j, :] - pa[j, :])
    return carry

  lax.fori_loop(0, _BPW, prodloop, 0)
  pltpu.sync_copy(prod, out_hbm.at[pl.ds(w * _BPW, _BPW)])


_final_kernel = functools.partial(
    pl.kernel,
    out_type=jax.ShapeDtypeStruct((_BATCH, _EMB), jnp.float32),
    compiler_params=pltpu.CompilerParams(use_tc_tiling_on_sc=False),
    mesh=plsc.VectorSubcoreMesh(core_axis_name="c", subcore_axis_name="s"),
    scratch_types=[
        pltpu.VMEM((_BPW,), jnp.int32),
        pltpu.VMEM((_BPW,), jnp.int32),
        pltpu.VMEM((_BPW,), jnp.int32),
        pltpu.VMEM((_BPW, _EMB), jnp.float32),
        pltpu.VMEM((_BPW, _EMB), jnp.float32),
        pltpu.VMEM((_BPW, _EMB), jnp.float32),
        pltpu.VMEM((_BPW, _EMB), jnp.float32),
        pltpu.VMEM((_BPW, _EMB), jnp.float32),
    ],
)(_final_body)


def _loss_body(prod_ref, out_ref):
  d = jnp.sum(prod_ref[...], axis=1) * (1.0 / (_N_LAYERS + 1) ** 2)
  sp = jnp.maximum(d, 0.0) + jnp.log(1.0 + jnp.exp(-jnp.abs(d)))
  out_ref[...] = jnp.mean(sp).reshape(1, 1)


def _loss(prod):
  return pl.pallas_call(
      _loss_body,
      out_shape=jax.ShapeDtypeStruct((1, 1), jnp.float32),
  )(prod)


def kernel(user_embedding, item_embedding, graph_values, edge_index,
           users, positive_items, negative_items):
  x = jnp.concatenate([user_embedding, item_embedding], axis=0)
  rows2d = edge_index[0].astype(jnp.int32).reshape(_NBLK, _BLK)
  cols2d = edge_index[1].astype(jnp.int32).reshape(_NBLK, _BLK)
  vals2d = graph_values.reshape(_NBLK, _BLK)
  users2d = users.astype(jnp.int32).reshape(_NC * _NS, _BPW)
  pos2d = positive_items.astype(jnp.int32).reshape(_NC * _NS, _BPW)
  neg2d = negative_items.astype(jnp.int32).reshape(_NC * _NS, _BPW)

  tables = [x]
  for _ in range(_N_LAYERS):
    x = _layer_kernel(x, rows2d, cols2d, vals2d)
    tables.append(x)

  prod = _final_kernel(tables[0], tables[1], tables[2], tables[3],
                       users2d, pos2d, neg2d)
  return _loss(prod)[0, 0]
